# baseline (speedup 1.0000x reference)
import functools

import jax
import jax.numpy as jnp
from jax import lax
from jax.experimental import pallas as pl
from jax.experimental.pallas import tpu as pltpu
from jax.experimental.pallas import tpu_sc as plsc

_N = 10000
_D = 128
_NP = 10240
_NW = 32
_NTILES = 16
_CHUNK = 128
_STRIPE = _NP // _NTILES
_RB = 1000
_NB = _N // _RB


def _make_deg_kernel(ep):
    per_tile = ep // _NW
    mesh = plsc.VectorSubcoreMesh(core_axis_name="c", subcore_axis_name="s")

    @functools.partial(
        pl.kernel,
        out_type=jax.ShapeDtypeStruct((_NW, _NP), jnp.float32),
        mesh=mesh,
        scratch_types=[
            pltpu.VMEM((per_tile,), jnp.int32),
            pltpu.VMEM((_NP,), jnp.float32),
        ],
        compiler_params=pltpu.CompilerParams(needs_layout_passes=False),
    )
    def deg_kernel(col_hbm, hist_hbm, col_v, table_v):
        c = lax.axis_index("c")
        s = lax.axis_index("s")
        w = c * _NTILES + s
        pltpu.sync_copy(col_hbm.at[pl.ds(w * per_tile, per_tile)], col_v)
        zeros = jnp.zeros((16,), jnp.float32)

        def zero_body(i, carry):
            table_v[pl.ds(i * 16, 16)] = zeros
            return carry

        lax.fori_loop(0, _NP // 16, zero_body, 0)
        ones = jnp.ones((16,), jnp.float32)

        def hist_body(i, carry):
            idx = col_v[pl.ds(i * 16, 16)]
            plsc.addupdate_scatter(table_v, [idx], ones)
            return carry

        lax.fori_loop(0, per_tile // 16, hist_body, 0)
        pltpu.sync_copy(table_v, hist_hbm.at[w])

    return deg_kernel


_G = 8


def _make_propagate_kernel(ep):
    chunks_pt = ep // (_NTILES * _CHUNK)
    pairs = chunks_pt // (2 * _G)
    mesh = plsc.VectorSubcoreMesh(core_axis_name="c", subcore_axis_name="s")

    @functools.partial(
        pl.kernel,
        out_type=jax.ShapeDtypeStruct((_NP, _D), jnp.float32),
        mesh=mesh,
        scratch_types=[
            pltpu.VMEM((2, _G, _CHUNK), jnp.int32),
            pltpu.VMEM((2, _G, _CHUNK), jnp.int32),
            pltpu.VMEM((_CHUNK, _D), jnp.float32),
            pltpu.VMEM((_CHUNK, _D), jnp.float32),
            pltpu.VMEM_SHARED((_NP, _D), jnp.float32),
            pltpu.SemaphoreType.DMA,
            pltpu.SemaphoreType.DMA,
            pltpu.SemaphoreType.DMA,
            pltpu.SemaphoreType.DMA,
        ],
        compiler_params=pltpu.CompilerParams(needs_layout_passes=False),
    )
    def prop_kernel(row_hbm, col_hbm, g_hbm, out_hbm, row_w, col_w, buf0, buf1,
                    agg_s, semg0, semg1, semi0, semi1):
        c = lax.axis_index("c")
        s = lax.axis_index("s")
        cb = s * chunks_pt
        bufs = (buf0, buf1)
        semgs = (semg0, semg1)
        semis = (semi0, semi1)

        def issue_idx(gstart, slot):
            pltpu.async_copy(row_hbm.at[pl.ds(cb + gstart, _G)], row_w.at[slot],
                             semis[slot])
            pltpu.async_copy(col_hbm.at[pl.ds(cb + gstart, _G)], col_w.at[slot],
                             semis[slot])

        def wait_idx(slot):
            pltpu.make_async_copy(row_hbm.at[pl.ds(0, _G)], row_w.at[slot],
                                  semis[slot]).wait()
            pltpu.make_async_copy(col_hbm.at[pl.ds(0, _G)], col_w.at[slot],
                                  semis[slot]).wait()

        def issue_gather(slot, u, p):
            pltpu.async_copy(g_hbm.at[row_w.at[slot, u]], bufs[p], semgs[p])

        def wait_gather(slot, u, p):
            pltpu.make_async_copy(g_hbm.at[row_w.at[slot, u]], bufs[p],
                                  semgs[p]).wait()

        @pl.when(c == 0)
        def _zero_phase():
            with jax.named_scope("agg_zero"):
                zeros = jnp.zeros((16,), jnp.float32)

                def zero_body(i, carry):
                    buf0[i // 8, pl.ds((i % 8) * 16, 16)] = zeros
                    return carry

                lax.fori_loop(0, _CHUNK * 8, zero_body, 0)
                for m in range(_STRIPE // _CHUNK):
                    pltpu.sync_copy(
                        buf0, agg_s.at[pl.ds(s * _STRIPE + m * _CHUNK,
                                             _CHUNK)])
            plsc.subcore_barrier()
            issue_idx(0, 0)
            wait_idx(0)
            issue_gather(0, 0, 0)
            issue_idx(_G, 1)

        def outer_body(j, carry):
            for u in range(2 * _G):
                slot = 0 if u < _G else 1
                uu = u % _G
                p = u % 2
                if u == _G - 1:
                    wait_idx(1)
                if u < 2 * _G - 1:
                    nslot = 0 if (u + 1) < _G else 1
                    issue_gather(nslot, (u + 1) % _G, (u + 1) % 2)
                else:
                    @pl.when(j + 1 < pairs)
                    def _next_pair_gather():
                        wait_idx(0)
                        issue_gather(0, 0, 0)
                wait_gather(slot, uu, p)
                pltpu.sync_copy(bufs[p], agg_s.at[col_w.at[slot, uu]],
                                add=True)
                if u == _G - 1:
                    @pl.when(j + 1 < pairs)
                    def _next_pair_idx():
                        issue_idx((j + 1) * 2 * _G, 0)
                if u == 2 * _G - 1:
                    @pl.when(j + 1 < pairs)
                    def _next_pair_idx2():
                        issue_idx((j + 1) * 2 * _G + _G, 1)
            return carry

        @pl.when(c == 0)
        def _edge_phase():
            with jax.named_scope("agg_edges"):
                lax.fori_loop(0, pairs, outer_body, 0)
            plsc.subcore_barrier()
            with jax.named_scope("agg_dump"):
                pltpu.sync_copy(agg_s.at[pl.ds(s * _STRIPE, _STRIPE)],
                                out_hbm.at[pl.ds(s * _STRIPE, _STRIPE)])

    return prop_kernel


def _dinv_body(hist_ref, dinv_ref):
    deg = jnp.sum(hist_ref[...], axis=0) + 1.0
    dinv_ref[...] = lax.rsqrt(deg).reshape(-1, 1)


def _dinv(hist):
    blk = 1024
    return pl.pallas_call(
        _dinv_body,
        out_shape=jax.ShapeDtypeStruct((_NP, 1), jnp.float32),
        grid=(_NP // blk,),
        in_specs=[pl.BlockSpec((_NW, blk), lambda b: (0, b))],
        out_specs=pl.BlockSpec((blk, 1), lambda b: (b, 0)),
    )(hist)


def _init_body(x_ref, wt_ref, b_ref, dinv_ref, h_ref, g_ref):
    h = jnp.dot(x_ref[...], wt_ref[...], preferred_element_type=jnp.float32)
    h = h + b_ref[...]
    h_ref[...] = h
    g_ref[...] = h * dinv_ref[...]


def _init_linear(x, fc_wt, fc_b, dinv):
    return pl.pallas_call(
        _init_body,
        out_shape=(
            jax.ShapeDtypeStruct((_N, _D), jnp.float32),
            jax.ShapeDtypeStruct((_N, _D), jnp.float32),
        ),
        grid=(_NB,),
        in_specs=[
            pl.BlockSpec((_RB, _D), lambda b: (b, 0)),
            pl.BlockSpec((_D, _D), lambda b: (0, 0)),
            pl.BlockSpec((1, _D), lambda b: (0, 0)),
            pl.BlockSpec((_RB, 1), lambda b: (b, 0)),
        ],
        out_specs=(
            pl.BlockSpec((_RB, _D), lambda b: (b, 0)),
            pl.BlockSpec((_RB, _D), lambda b: (b, 0)),
        ),
    )(x, fc_wt, fc_b, dinv)


def _layer_body(alpha, p_ref, g_ref, h0_ref, dinv_ref, w_ref, gn_ref):
    dinv = dinv_ref[...]
    agg = (p_ref[...] + g_ref[...]) * dinv
    out = agg * (1.0 - alpha) + alpha * h0_ref[...]
    t = jnp.dot(out, w_ref[...], preferred_element_type=jnp.float32)
    hn = jnp.where(t > 0, t, jnp.exp(jnp.minimum(t, 0.0)) - 1.0)
    gn_ref[...] = hn * dinv


def _layer(alpha, p, g, h0, dinv, w):
    return pl.pallas_call(
        functools.partial(_layer_body, alpha),
        out_shape=jax.ShapeDtypeStruct((_N, _D), jnp.float32),
        grid=(_NB,),
        in_specs=[
            pl.BlockSpec((_RB, _D), lambda b: (b, 0)),
            pl.BlockSpec((_RB, _D), lambda b: (b, 0)),
            pl.BlockSpec((_RB, _D), lambda b: (b, 0)),
            pl.BlockSpec((_RB, 1), lambda b: (b, 0)),
            pl.BlockSpec((_D, _D), lambda b: (0, 0)),
        ],
        out_specs=pl.BlockSpec((_RB, _D), lambda b: (b, 0)),
    )(p, g, h0, dinv, w)


def _final_body(alpha, p_ref, g_ref, h0_ref, dinv_ref, w_ref, owt_ref, ob_ref,
                y_ref):
    dinv = dinv_ref[...]
    agg = (p_ref[...] + g_ref[...]) * dinv
    out = agg * (1.0 - alpha) + alpha * h0_ref[...]
    t = jnp.dot(out, w_ref[...], preferred_element_type=jnp.float32)
    hn = jnp.where(t > 0, t, jnp.exp(jnp.minimum(t, 0.0)) - 1.0)
    y = jnp.dot(hn, owt_ref[...], preferred_element_type=jnp.float32)
    y_ref[...] = y + ob_ref[...]


def _final_layer(alpha, p, g, h0, dinv, w, out_wt, out_b):
    return pl.pallas_call(
        functools.partial(_final_body, alpha),
        out_shape=jax.ShapeDtypeStruct((_N, _D), jnp.float32),
        grid=(_NB,),
        in_specs=[
            pl.BlockSpec((_RB, _D), lambda b: (b, 0)),
            pl.BlockSpec((_RB, _D), lambda b: (b, 0)),
            pl.BlockSpec((_RB, _D), lambda b: (b, 0)),
            pl.BlockSpec((_RB, 1), lambda b: (b, 0)),
            pl.BlockSpec((_D, _D), lambda b: (0, 0)),
            pl.BlockSpec((_D, _D), lambda b: (0, 0)),
            pl.BlockSpec((1, _D), lambda b: (0, 0)),
        ],
        out_specs=pl.BlockSpec((_RB, _D), lambda b: (b, 0)),
    )(p, g, h0, dinv, w, out_wt, out_b)


def kernel(x, edge_index, fc_w, fc_b, w0, w1, w2, w3, out_w, out_b):
    e = edge_index.shape[1]
    unit = _NTILES * 2 * _G * _CHUNK
    ep = -(-e // unit) * unit
    pad = ep - e
    row = jnp.concatenate([edge_index[0], jnp.zeros((pad,), jnp.int32)])
    col = jnp.concatenate([edge_index[1], jnp.full((pad,), _N, jnp.int32)])
    row3 = row.reshape(ep // _CHUNK, _CHUNK)
    col3 = col.reshape(ep // _CHUNK, _CHUNK)

    deg_kernel = _make_deg_kernel(ep)
    prop_kernel = _make_propagate_kernel(ep)

    hist = deg_kernel(col)
    dinv = _dinv(hist)[:_N]
    h0, g = _init_linear(x, fc_w.T, fc_b.reshape(1, _D), dinv)

    ws = [w0, w1, w2, w3]
    for i in range(3):
        p = prop_kernel(row3, col3, g)
        g = _layer(i / 4.0, p, g, h0, dinv, ws[i])
    p = prop_kernel(row3, col3, g)
    return _final_layer(3 / 4.0, p, g, h0, dinv, ws[3], out_w.T,
                        out_b.reshape(1, _D))


# speedup vs baseline: 4.2350x; 4.2350x over previous
import functools

import jax
import jax.numpy as jnp
from jax import lax
from jax.experimental import pallas as pl
from jax.experimental.pallas import tpu as pltpu
from jax.experimental.pallas import tpu_sc as plsc

_N = 10000
_D = 128
_NP = 10240
_NW = 32
_NTILES = 16
_CHUNK = 128
_STRIPE = _NP // _NTILES
_RB = 1000
_NB = _N // _RB


def _make_deg_kernel(ep):
    per_tile = ep // _NW
    mesh = plsc.VectorSubcoreMesh(core_axis_name="c", subcore_axis_name="s")

    @functools.partial(
        pl.kernel,
        out_type=jax.ShapeDtypeStruct((_NW, _NP), jnp.float32),
        mesh=mesh,
        scratch_types=[
            pltpu.VMEM((per_tile,), jnp.int32),
            pltpu.VMEM((_NP,), jnp.float32),
        ],
        compiler_params=pltpu.CompilerParams(needs_layout_passes=False),
    )
    def deg_kernel(col_hbm, hist_hbm, col_v, table_v):
        c = lax.axis_index("c")
        s = lax.axis_index("s")
        w = c * _NTILES + s
        pltpu.sync_copy(col_hbm.at[pl.ds(w * per_tile, per_tile)], col_v)
        zeros = jnp.zeros((16,), jnp.float32)

        def zero_body(i, carry):
            table_v[pl.ds(i * 16, 16)] = zeros
            return carry

        lax.fori_loop(0, _NP // 16, zero_body, 0)
        ones = jnp.ones((16,), jnp.float32)

        def hist_body(i, carry):
            idx = col_v[pl.ds(i * 16, 16)]
            plsc.addupdate_scatter(table_v, [idx], ones)
            return carry

        lax.fori_loop(0, per_tile // 16, hist_body, 0)
        pltpu.sync_copy(table_v, hist_hbm.at[w])

    return deg_kernel


_G = 8


def _make_propagate_kernel(ep):
    chunks_pt = ep // (_NW * _CHUNK)
    pairs = chunks_pt // (2 * _G)
    mesh = plsc.VectorSubcoreMesh(core_axis_name="c", subcore_axis_name="s")

    @functools.partial(
        pl.kernel,
        out_type=jax.ShapeDtypeStruct((2, _NP, _D), jnp.float32),
        mesh=mesh,
        scratch_types=[
            pltpu.VMEM((2, _G, _CHUNK), jnp.int32),
            pltpu.VMEM((2, _G, _CHUNK), jnp.int32),
            pltpu.VMEM((_CHUNK, _D), jnp.float32),
            pltpu.VMEM((_CHUNK, _D), jnp.float32),
            pltpu.VMEM_SHARED((_NP, _D), jnp.float32),
            pltpu.SemaphoreType.DMA,
            pltpu.SemaphoreType.DMA,
            pltpu.SemaphoreType.DMA,
            pltpu.SemaphoreType.DMA,
        ],
        compiler_params=pltpu.CompilerParams(needs_layout_passes=False),
    )
    def prop_kernel(row_hbm, col_hbm, g_hbm, out_hbm, row_w, col_w, buf0, buf1,
                    agg_s, semg0, semg1, semi0, semi1):
        c = lax.axis_index("c")
        s = lax.axis_index("s")
        cb = (c * _NTILES + s) * chunks_pt
        bufs = (buf0, buf1)
        semgs = (semg0, semg1)
        semis = (semi0, semi1)

        def issue_idx(gstart, slot):
            pltpu.async_copy(row_hbm.at[pl.ds(cb + gstart, _G)], row_w.at[slot],
                             semis[slot])
            pltpu.async_copy(col_hbm.at[pl.ds(cb + gstart, _G)], col_w.at[slot],
                             semis[slot])

        def wait_idx(slot):
            pltpu.make_async_copy(row_hbm.at[pl.ds(0, _G)], row_w.at[slot],
                                  semis[slot]).wait()
            pltpu.make_async_copy(col_hbm.at[pl.ds(0, _G)], col_w.at[slot],
                                  semis[slot]).wait()

        def issue_gather(slot, u, p):
            pltpu.async_copy(g_hbm.at[row_w.at[slot, u]], bufs[p], semgs[p])

        def wait_gather(slot, u, p):
            pltpu.make_async_copy(g_hbm.at[row_w.at[slot, u]], bufs[p],
                                  semgs[p]).wait()

        with jax.named_scope("agg_zero"):
            zeros = jnp.zeros((16,), jnp.float32)

            def zero_body(i, carry):
                buf0[i // 8, pl.ds((i % 8) * 16, 16)] = zeros
                return carry

            lax.fori_loop(0, _CHUNK * 8, zero_body, 0)
            for m in range(_STRIPE // _CHUNK):
                pltpu.sync_copy(
                    buf0, agg_s.at[pl.ds(s * _STRIPE + m * _CHUNK, _CHUNK)])
        plsc.subcore_barrier()
        issue_idx(0, 0)
        wait_idx(0)
        issue_gather(0, 0, 0)
        issue_idx(_G, 1)

        def outer_body(j, carry):
            for u in range(2 * _G):
                slot = 0 if u < _G else 1
                uu = u % _G
                p = u % 2
                if u == _G - 1:
                    wait_idx(1)
                if u < 2 * _G - 1:
                    nslot = 0 if (u + 1) < _G else 1
                    issue_gather(nslot, (u + 1) % _G, (u + 1) % 2)
                else:
                    @pl.when(j + 1 < pairs)
                    def _next_pair_gather():
                        wait_idx(0)
                        issue_gather(0, 0, 0)
                wait_gather(slot, uu, p)
                pltpu.sync_copy(bufs[p], agg_s.at[col_w.at[slot, uu]],
                                add=True)
                if u == _G - 1:
                    @pl.when(j + 1 < pairs)
                    def _next_pair_idx():
                        issue_idx((j + 1) * 2 * _G, 0)
                if u == 2 * _G - 1:
                    @pl.when(j + 1 < pairs)
                    def _next_pair_idx2():
                        issue_idx((j + 1) * 2 * _G + _G, 1)
            return carry

        with jax.named_scope("agg_edges"):
            lax.fori_loop(0, pairs, outer_body, 0)
        plsc.subcore_barrier()
        with jax.named_scope("agg_dump"):
            pltpu.sync_copy(agg_s.at[pl.ds(s * _STRIPE, _STRIPE)],
                            out_hbm.at[c, pl.ds(s * _STRIPE, _STRIPE)])

    return prop_kernel


def _dinv_body(hist_ref, dinv_ref):
    deg = jnp.sum(hist_ref[...], axis=0) + 1.0
    dinv_ref[...] = lax.rsqrt(deg).reshape(-1, 1)


def _dinv(hist):
    blk = 1024
    return pl.pallas_call(
        _dinv_body,
        out_shape=jax.ShapeDtypeStruct((_NP, 1), jnp.float32),
        grid=(_NP // blk,),
        in_specs=[pl.BlockSpec((_NW, blk), lambda b: (0, b))],
        out_specs=pl.BlockSpec((blk, 1), lambda b: (b, 0)),
    )(hist)


def _init_body(x_ref, wt_ref, b_ref, dinv_ref, h_ref, g_ref):
    h = jnp.dot(x_ref[...], wt_ref[...], preferred_element_type=jnp.float32)
    h = h + b_ref[...]
    h_ref[...] = h
    g_ref[...] = h * dinv_ref[...]


def _init_linear(x, fc_wt, fc_b, dinv):
    return pl.pallas_call(
        _init_body,
        out_shape=(
            jax.ShapeDtypeStruct((_N, _D), jnp.float32),
            jax.ShapeDtypeStruct((_N, _D), jnp.float32),
        ),
        grid=(_NB,),
        in_specs=[
            pl.BlockSpec((_RB, _D), lambda b: (b, 0)),
            pl.BlockSpec((_D, _D), lambda b: (0, 0)),
            pl.BlockSpec((1, _D), lambda b: (0, 0)),
            pl.BlockSpec((_RB, 1), lambda b: (b, 0)),
        ],
        out_specs=(
            pl.BlockSpec((_RB, _D), lambda b: (b, 0)),
            pl.BlockSpec((_RB, _D), lambda b: (b, 0)),
        ),
    )(x, fc_wt, fc_b, dinv)


def _layer_body(alpha, p_ref, g_ref, h0_ref, dinv_ref, w_ref, gn_ref):
    dinv = dinv_ref[...]
    agg = (p_ref[0] + p_ref[1] + g_ref[...]) * dinv
    out = agg * (1.0 - alpha) + alpha * h0_ref[...]
    t = jnp.dot(out, w_ref[...], preferred_element_type=jnp.float32)
    hn = jnp.where(t > 0, t, jnp.exp(jnp.minimum(t, 0.0)) - 1.0)
    gn_ref[...] = hn * dinv


def _layer(alpha, p, g, h0, dinv, w):
    return pl.pallas_call(
        functools.partial(_layer_body, alpha),
        out_shape=jax.ShapeDtypeStruct((_N, _D), jnp.float32),
        grid=(_NB,),
        in_specs=[
            pl.BlockSpec((2, _RB, _D), lambda b: (0, b, 0)),
            pl.BlockSpec((_RB, _D), lambda b: (b, 0)),
            pl.BlockSpec((_RB, _D), lambda b: (b, 0)),
            pl.BlockSpec((_RB, 1), lambda b: (b, 0)),
            pl.BlockSpec((_D, _D), lambda b: (0, 0)),
        ],
        out_specs=pl.BlockSpec((_RB, _D), lambda b: (b, 0)),
    )(p, g, h0, dinv, w)


def _final_body(alpha, p_ref, g_ref, h0_ref, dinv_ref, w_ref, owt_ref, ob_ref,
                y_ref):
    dinv = dinv_ref[...]
    agg = (p_ref[0] + p_ref[1] + g_ref[...]) * dinv
    out = agg * (1.0 - alpha) + alpha * h0_ref[...]
    t = jnp.dot(out, w_ref[...], preferred_element_type=jnp.float32)
    hn = jnp.where(t > 0, t, jnp.exp(jnp.minimum(t, 0.0)) - 1.0)
    y = jnp.dot(hn, owt_ref[...], preferred_element_type=jnp.float32)
    y_ref[...] = y + ob_ref[...]


def _final_layer(alpha, p, g, h0, dinv, w, out_wt, out_b):
    return pl.pallas_call(
        functools.partial(_final_body, alpha),
        out_shape=jax.ShapeDtypeStruct((_N, _D), jnp.float32),
        grid=(_NB,),
        in_specs=[
            pl.BlockSpec((2, _RB, _D), lambda b: (0, b, 0)),
            pl.BlockSpec((_RB, _D), lambda b: (b, 0)),
            pl.BlockSpec((_RB, _D), lambda b: (b, 0)),
            pl.BlockSpec((_RB, 1), lambda b: (b, 0)),
            pl.BlockSpec((_D, _D), lambda b: (0, 0)),
            pl.BlockSpec((_D, _D), lambda b: (0, 0)),
            pl.BlockSpec((1, _D), lambda b: (0, 0)),
        ],
        out_specs=pl.BlockSpec((_RB, _D), lambda b: (b, 0)),
    )(p, g, h0, dinv, w, out_wt, out_b)


def kernel(x, edge_index, fc_w, fc_b, w0, w1, w2, w3, out_w, out_b):
    e = edge_index.shape[1]
    unit = _NW * 2 * _G * _CHUNK
    ep = -(-e // unit) * unit
    pad = ep - e
    pad_iota = jnp.arange(pad, dtype=jnp.int32)
    row = jnp.concatenate([edge_index[0], pad_iota % 256])
    col = jnp.concatenate([edge_index[1], _N + pad_iota % (_NP - _N)])
    row3 = row.reshape(ep // _CHUNK, _CHUNK)
    col3 = col.reshape(ep // _CHUNK, _CHUNK)

    deg_kernel = _make_deg_kernel(ep)
    prop_kernel = _make_propagate_kernel(ep)

    hist = deg_kernel(col)
    dinv = _dinv(hist)[:_N]
    h0, g = _init_linear(x, fc_w.T, fc_b.reshape(1, _D), dinv)

    ws = [w0, w1, w2, w3]
    for i in range(3):
        p = prop_kernel(row3, col3, g)
        g = _layer(i / 4.0, p, g, h0, dinv, ws[i])
    p = prop_kernel(row3, col3, g)
    return _final_layer(3 / 4.0, p, g, h0, dinv, ws[3], out_w.T,
                        out_b.reshape(1, _D))


# speedup vs baseline: 4.3513x; 1.0275x over previous
import functools

import jax
import jax.numpy as jnp
from jax import lax
from jax.experimental import pallas as pl
from jax.experimental.pallas import tpu as pltpu
from jax.experimental.pallas import tpu_sc as plsc

_N = 10000
_D = 128
_NP = 10240
_NW = 32
_NTILES = 16
_CHUNK = 128
_STRIPE = _NP // _NTILES
_RB = 1000
_NB = _N // _RB


def _make_deg_kernel(ep):
    per_tile = ep // _NW
    mesh = plsc.VectorSubcoreMesh(core_axis_name="c", subcore_axis_name="s")

    @functools.partial(
        pl.kernel,
        out_type=jax.ShapeDtypeStruct((_NW, _NP), jnp.float32),
        mesh=mesh,
        scratch_types=[
            pltpu.VMEM((per_tile,), jnp.int32),
            pltpu.VMEM((_NP,), jnp.float32),
        ],
        compiler_params=pltpu.CompilerParams(needs_layout_passes=False),
    )
    def deg_kernel(col_hbm, hist_hbm, col_v, table_v):
        c = lax.axis_index("c")
        s = lax.axis_index("s")
        w = c * _NTILES + s
        pltpu.sync_copy(col_hbm.at[pl.ds(w * per_tile, per_tile)], col_v)
        zeros = jnp.zeros((16,), jnp.float32)

        def zero_body(i, carry):
            table_v[pl.ds(i * 16, 16)] = zeros
            return carry

        lax.fori_loop(0, _NP // 16, zero_body, 0)
        ones = jnp.ones((16,), jnp.float32)

        def hist_body(i, carry):
            idx = col_v[pl.ds(i * 16, 16)]
            plsc.addupdate_scatter(table_v, [idx], ones)
            return carry

        lax.fori_loop(0, per_tile // 16, hist_body, 0)
        pltpu.sync_copy(table_v, hist_hbm.at[w])

    return deg_kernel


_G = 8


def _make_propagate_kernel(ep):
    chunks_pt = ep // (_NW * _CHUNK)
    pairs = chunks_pt // (2 * _G)
    mesh = plsc.VectorSubcoreMesh(core_axis_name="c", subcore_axis_name="s")

    @functools.partial(
        pl.kernel,
        out_type=jax.ShapeDtypeStruct((2, _NP, _D), jnp.float32),
        mesh=mesh,
        scratch_types=[
            pltpu.VMEM((2, _G, _CHUNK), jnp.int32),
            pltpu.VMEM((2, _G, _CHUNK), jnp.int32),
            pltpu.VMEM((_CHUNK, _D), jnp.float32),
            pltpu.VMEM((_CHUNK, _D), jnp.float32),
            pltpu.VMEM_SHARED((_NP, _D), jnp.float32),
            pltpu.SemaphoreType.DMA,
            pltpu.SemaphoreType.DMA,
            pltpu.SemaphoreType.DMA,
            pltpu.SemaphoreType.DMA,
        ],
        compiler_params=pltpu.CompilerParams(needs_layout_passes=False),
    )
    def prop_kernel(row_hbm, col_hbm, g_hbm, out_hbm, row_w, col_w, buf0, buf1,
                    agg_s, semg0, semg1, semi0, semi1):
        c = lax.axis_index("c")
        s = lax.axis_index("s")
        cb = (c * _NTILES + s) * chunks_pt
        bufs = (buf0, buf1)
        semgs = (semg0, semg1)
        semis = (semi0, semi1)

        def issue_idx(gstart, slot):
            pltpu.async_copy(row_hbm.at[pl.ds(cb + gstart, _G)], row_w.at[slot],
                             semis[slot])
            pltpu.async_copy(col_hbm.at[pl.ds(cb + gstart, _G)], col_w.at[slot],
                             semis[slot])

        def wait_idx(slot):
            pltpu.make_async_copy(row_hbm.at[pl.ds(0, _G)], row_w.at[slot],
                                  semis[slot]).wait()
            pltpu.make_async_copy(col_hbm.at[pl.ds(0, _G)], col_w.at[slot],
                                  semis[slot]).wait()

        def issue_gather(slot, u, p):
            pltpu.async_copy(g_hbm.at[row_w.at[slot, u]], bufs[p], semgs[p])

        def wait_gather(slot, u, p):
            pltpu.make_async_copy(g_hbm.at[row_w.at[slot, u]], bufs[p],
                                  semgs[p]).wait()

        with jax.named_scope("agg_zero"):
            zeros = jnp.zeros((16,), jnp.float32)

            def zero_body(i, carry):
                for l in range(8):
                    buf0[i, pl.ds(l * 16, 16)] = zeros
                return carry

            lax.fori_loop(0, _CHUNK, zero_body, 0)
            for m in range(_STRIPE // _CHUNK):
                pltpu.sync_copy(
                    buf0, agg_s.at[pl.ds(s * _STRIPE + m * _CHUNK, _CHUNK)])
        plsc.subcore_barrier()
        issue_idx(0, 0)
        wait_idx(0)
        issue_gather(0, 0, 0)
        issue_idx(_G, 1)

        def outer_body(j, carry):
            for u in range(2 * _G):
                slot = 0 if u < _G else 1
                uu = u % _G
                p = u % 2
                if u == _G - 1:
                    wait_idx(1)
                if u < 2 * _G - 1:
                    nslot = 0 if (u + 1) < _G else 1
                    issue_gather(nslot, (u + 1) % _G, (u + 1) % 2)
                else:
                    @pl.when(j + 1 < pairs)
                    def _next_pair_gather():
                        wait_idx(0)
                        issue_gather(0, 0, 0)
                wait_gather(slot, uu, p)
                pltpu.sync_copy(bufs[p], agg_s.at[col_w.at[slot, uu]],
                                add=True)
                if u == _G - 1:
                    @pl.when(j + 1 < pairs)
                    def _next_pair_idx():
                        issue_idx((j + 1) * 2 * _G, 0)
                if u == 2 * _G - 1:
                    @pl.when(j + 1 < pairs)
                    def _next_pair_idx2():
                        issue_idx((j + 1) * 2 * _G + _G, 1)
            return carry

        with jax.named_scope("agg_edges"):
            lax.fori_loop(0, pairs, outer_body, 0)
        plsc.subcore_barrier()
        with jax.named_scope("agg_dump"):
            pltpu.sync_copy(agg_s.at[pl.ds(s * _STRIPE, _STRIPE)],
                            out_hbm.at[c, pl.ds(s * _STRIPE, _STRIPE)])

    return prop_kernel


def _dinv_body(hist_ref, dinv_ref):
    deg = jnp.sum(hist_ref[...], axis=0) + 1.0
    dinv_ref[...] = lax.rsqrt(deg).reshape(-1, 1)


def _dinv(hist):
    blk = 1024
    return pl.pallas_call(
        _dinv_body,
        out_shape=jax.ShapeDtypeStruct((_NP, 1), jnp.float32),
        grid=(_NP // blk,),
        in_specs=[pl.BlockSpec((_NW, blk), lambda b: (0, b))],
        out_specs=pl.BlockSpec((blk, 1), lambda b: (b, 0)),
    )(hist)


def _init_body(x_ref, wt_ref, b_ref, dinv_ref, h_ref, g_ref):
    h = jnp.dot(x_ref[...], wt_ref[...], preferred_element_type=jnp.float32)
    h = h + b_ref[...]
    h_ref[...] = h
    g_ref[...] = h * dinv_ref[...]


def _init_linear(x, fc_wt, fc_b, dinv):
    return pl.pallas_call(
        _init_body,
        out_shape=(
            jax.ShapeDtypeStruct((_N, _D), jnp.float32),
            jax.ShapeDtypeStruct((_N, _D), jnp.float32),
        ),
        grid=(_NB,),
        in_specs=[
            pl.BlockSpec((_RB, _D), lambda b: (b, 0)),
            pl.BlockSpec((_D, _D), lambda b: (0, 0)),
            pl.BlockSpec((1, _D), lambda b: (0, 0)),
            pl.BlockSpec((_RB, 1), lambda b: (b, 0)),
        ],
        out_specs=(
            pl.BlockSpec((_RB, _D), lambda b: (b, 0)),
            pl.BlockSpec((_RB, _D), lambda b: (b, 0)),
        ),
    )(x, fc_wt, fc_b, dinv)


def _layer_body(alpha, p_ref, g_ref, h0_ref, dinv_ref, w_ref, gn_ref):
    dinv = dinv_ref[...]
    agg = (p_ref[0] + p_ref[1] + g_ref[...]) * dinv
    out = agg * (1.0 - alpha) + alpha * h0_ref[...]
    t = jnp.dot(out, w_ref[...], preferred_element_type=jnp.float32)
    hn = jnp.where(t > 0, t, jnp.exp(jnp.minimum(t, 0.0)) - 1.0)
    gn_ref[...] = hn * dinv


def _layer(alpha, p, g, h0, dinv, w):
    return pl.pallas_call(
        functools.partial(_layer_body, alpha),
        out_shape=jax.ShapeDtypeStruct((_N, _D), jnp.float32),
        grid=(_NB,),
        in_specs=[
            pl.BlockSpec((2, _RB, _D), lambda b: (0, b, 0)),
            pl.BlockSpec((_RB, _D), lambda b: (b, 0)),
            pl.BlockSpec((_RB, _D), lambda b: (b, 0)),
            pl.BlockSpec((_RB, 1), lambda b: (b, 0)),
            pl.BlockSpec((_D, _D), lambda b: (0, 0)),
        ],
        out_specs=pl.BlockSpec((_RB, _D), lambda b: (b, 0)),
    )(p, g, h0, dinv, w)


def _final_body(alpha, p_ref, g_ref, h0_ref, dinv_ref, w_ref, owt_ref, ob_ref,
                y_ref):
    dinv = dinv_ref[...]
    agg = (p_ref[0] + p_ref[1] + g_ref[...]) * dinv
    out = agg * (1.0 - alpha) + alpha * h0_ref[...]
    t = jnp.dot(out, w_ref[...], preferred_element_type=jnp.float32)
    hn = jnp.where(t > 0, t, jnp.exp(jnp.minimum(t, 0.0)) - 1.0)
    y = jnp.dot(hn, owt_ref[...], preferred_element_type=jnp.float32)
    y_ref[...] = y + ob_ref[...]


def _final_layer(alpha, p, g, h0, dinv, w, out_wt, out_b):
    return pl.pallas_call(
        functools.partial(_final_body, alpha),
        out_shape=jax.ShapeDtypeStruct((_N, _D), jnp.float32),
        grid=(_NB,),
        in_specs=[
            pl.BlockSpec((2, _RB, _D), lambda b: (0, b, 0)),
            pl.BlockSpec((_RB, _D), lambda b: (b, 0)),
            pl.BlockSpec((_RB, _D), lambda b: (b, 0)),
            pl.BlockSpec((_RB, 1), lambda b: (b, 0)),
            pl.BlockSpec((_D, _D), lambda b: (0, 0)),
            pl.BlockSpec((_D, _D), lambda b: (0, 0)),
            pl.BlockSpec((1, _D), lambda b: (0, 0)),
        ],
        out_specs=pl.BlockSpec((_RB, _D), lambda b: (b, 0)),
    )(p, g, h0, dinv, w, out_wt, out_b)


def kernel(x, edge_index, fc_w, fc_b, w0, w1, w2, w3, out_w, out_b):
    e = edge_index.shape[1]
    unit = _NW * 2 * _G * _CHUNK
    ep = -(-e // unit) * unit
    pad = ep - e
    pad_iota = jnp.arange(pad, dtype=jnp.int32)
    row = jnp.concatenate([edge_index[0], pad_iota % 256])
    col = jnp.concatenate([edge_index[1], _N + pad_iota % (_NP - _N)])
    row3 = row.reshape(ep // _CHUNK, _CHUNK)
    col3 = col.reshape(ep // _CHUNK, _CHUNK)

    deg_kernel = _make_deg_kernel(ep)
    prop_kernel = _make_propagate_kernel(ep)

    hist = deg_kernel(col)
    dinv = _dinv(hist)[:_N]
    h0, g = _init_linear(x, fc_w.T, fc_b.reshape(1, _D), dinv)

    ws = [w0, w1, w2, w3]
    for i in range(3):
        p = prop_kernel(row3, col3, g)
        g = _layer(i / 4.0, p, g, h0, dinv, ws[i])
    p = prop_kernel(row3, col3, g)
    return _final_layer(3 / 4.0, p, g, h0, dinv, ws[3], out_w.T,
                        out_b.reshape(1, _D))
